# baseline (speedup 1.0000x reference)
import functools

import jax
import jax.numpy as jnp
from jax import lax
from jax.experimental import pallas as pl
from jax.experimental.pallas import tpu as pltpu
from jax.experimental.pallas import tpu_sc as plsc

_BUFFER_SIZE = 4194304
_BUF_LEN = 4195840
_TAIL = _BUF_LEN - _BUFFER_SIZE

_NC = 2
_NS = 16
_NW = _NC * _NS
_CHUNK = _BUFFER_SIZE // _NW


@functools.partial(
    pl.kernel,
    mesh=plsc.VectorSubcoreMesh(core_axis_name="c", subcore_axis_name="s"),
    out_type=jax.ShapeDtypeStruct((_BUF_LEN,), jnp.float32),
)
def _roll_overwrite(x_hbm, buf_hbm, out_hbm):
    cid = lax.axis_index("c")
    sid = lax.axis_index("s")
    wid = sid * _NC + cid
    base = wid * _CHUNK
    pltpu.sync_copy(
        x_hbm.at[pl.ds(base, _CHUNK)],
        out_hbm.at[pl.ds(_TAIL + base, _CHUNK)],
    )

    @pl.when(wid == 0)
    def _():
        pltpu.sync_copy(
            buf_hbm.at[pl.ds(_BUFFER_SIZE, _TAIL)],
            out_hbm.at[pl.ds(0, _TAIL)],
        )


def kernel(x, buffer):
    out = _roll_overwrite(x.reshape(_BUFFER_SIZE), buffer.reshape(_BUF_LEN))
    return out.reshape(1, _BUF_LEN)


# speedup vs baseline: 2.8540x; 2.8540x over previous
import functools

import jax
import jax.numpy as jnp
from jax import lax
from jax.experimental import pallas as pl
from jax.experimental.pallas import tpu as pltpu
from jax.experimental.pallas import tpu_sc as plsc

_BUFFER_SIZE = 4194304
_BUF_LEN = 4195840
_TAIL = _BUF_LEN - _BUFFER_SIZE

_NC = 2
_NS = 16
_NW = _NC * _NS
_CHUNK = _BUFFER_SIZE // _NW

_PIECE = 16384
_NB = 4
_NPIECE = _CHUNK // _PIECE


@functools.partial(
    pl.kernel,
    mesh=plsc.VectorSubcoreMesh(core_axis_name="c", subcore_axis_name="s"),
    out_type=jax.ShapeDtypeStruct((_BUF_LEN,), jnp.float32),
    scratch_types=[pltpu.VMEM((_NB, _PIECE), jnp.float32)]
    + [pltpu.SemaphoreType.DMA] * (2 * _NB),
)
def _roll_overwrite(x_hbm, buf_hbm, out_hbm, vbuf, *sems):
    cid = lax.axis_index("c")
    sid = lax.axis_index("s")
    wid = sid * _NC + cid
    base = wid * _CHUNK
    sin, sout = sems[:_NB], sems[_NB:]

    def in_copy(i):
        return pltpu.make_async_copy(
            x_hbm.at[pl.ds(base + i * _PIECE, _PIECE)],
            vbuf.at[i % _NB],
            sin[i % _NB],
        )

    def out_copy(i):
        return pltpu.make_async_copy(
            vbuf.at[i % _NB],
            out_hbm.at[pl.ds(_TAIL + base + i * _PIECE, _PIECE)],
            sout[i % _NB],
        )

    for i in range(min(_NB, _NPIECE)):
        in_copy(i).start()
    for i in range(_NPIECE):
        in_copy(i).wait()
        out_copy(i).start()
        if i + _NB < _NPIECE:
            out_copy(i).wait()
            in_copy(i + _NB).start()
    for i in range(max(0, _NPIECE - _NB), _NPIECE):
        out_copy(i).wait()

    @pl.when(wid == 0)
    def _():
        pltpu.sync_copy(
            buf_hbm.at[pl.ds(_BUFFER_SIZE, _TAIL)],
            out_hbm.at[pl.ds(0, _TAIL)],
        )


def kernel(x, buffer):
    out = _roll_overwrite(x.reshape(_BUFFER_SIZE), buffer.reshape(_BUF_LEN))
    return out.reshape(1, _BUF_LEN)
